# baseline (speedup 1.0000x reference)
import functools

import jax
import jax.numpy as jnp
from jax import lax
from jax.experimental import pallas as pl
from jax.experimental.pallas import tpu as pltpu
from jax.experimental.pallas import tpu_sc as plsc

_N = 8192
_NC = 2
_NS = 16
_NW = _NC * _NS
_RPW = _N // _NW
_CR = 2
_NCHUNK = _RPW // _CR
_NBUF = 4


def _sc_body(x_hbm, v_hbm, out_hbm, buf, vals, in_sems, out_sems):
    wid = lax.axis_index("c") * _NS + lax.axis_index("s")
    r0 = wid * _RPW

    pltpu.make_async_copy(
        v_hbm.at[pl.ds(r0, _RPW)], vals.at[pl.ds(0, _RPW)], in_sems.at[0]
    ).start()
    pltpu.make_async_copy(
        v_hbm.at[pl.ds(r0, _RPW)], vals.at[pl.ds(0, _RPW)], in_sems.at[0]
    ).wait()

    def start_in(c, b):
        pltpu.make_async_copy(
            x_hbm.at[pl.ds(r0 + c * _CR, _CR), :], buf.at[b], in_sems.at[b]
        ).start()

    def wait_in(b):
        pltpu.make_async_copy(
            x_hbm.at[pl.ds(r0, _CR), :], buf.at[b], in_sems.at[b]
        ).wait()

    def start_out(c, b):
        pltpu.make_async_copy(
            buf.at[b], out_hbm.at[pl.ds(r0 + c * _CR, _CR), :], out_sems.at[b]
        ).start()

    def wait_out(b):
        pltpu.make_async_copy(
            buf.at[b], out_hbm.at[pl.ds(r0, _CR), :], out_sems.at[b]
        ).wait()

    for b in range(_NBUF - 1):
        start_in(b, b)

    iota = lax.broadcasted_iota(jnp.int32, (16,), 0)
    diag_mask = iota < _CR

    def outer(o, _):
        for b in range(_NBUF):
            c = o * _NBUF + b
            wait_in(b)
            vals_v = plsc.load_gather(vals, [c * _CR + iota])
            col0 = r0 + c * _CR
            plsc.addupdate_scatter(
                buf.at[b], [iota, col0 + iota], vals_v, mask=diag_mask
            )
            start_out(c, b)
            nb = (b + _NBUF - 1) % _NBUF

            @pl.when(c + _NBUF - 1 < _NCHUNK)
            def _():
                @pl.when(c >= 1)
                def _():
                    wait_out(nb)

                start_in(c + _NBUF - 1, nb)

        return ()

    lax.fori_loop(0, _NCHUNK // _NBUF, outer, ())

    for b in range(_NBUF):
        wait_out(b)


def kernel(x, values):
    mesh = plsc.VectorSubcoreMesh(
        core_axis_name="c", subcore_axis_name="s", num_cores=_NC, num_subcores=_NS
    )
    f = pl.kernel(
        _sc_body,
        out_type=jax.ShapeDtypeStruct((_N, _N), jnp.float32),
        mesh=mesh,
        scratch_types=[
            pltpu.VMEM((_NBUF, _CR, _N), jnp.float32),
            pltpu.VMEM((_RPW + 16, ), jnp.float32),
            pltpu.SemaphoreType.DMA((_NBUF,)),
            pltpu.SemaphoreType.DMA((_NBUF,)),
        ],
        compiler_params=pltpu.CompilerParams(needs_layout_passes=False),
    )
    return f(x, values)


# speedup vs baseline: 1.0012x; 1.0012x over previous
import functools

import jax
import jax.numpy as jnp
from jax import lax
from jax.experimental import pallas as pl
from jax.experimental.pallas import tpu as pltpu
from jax.experimental.pallas import tpu_sc as plsc

_N = 8192
_NC = 2
_NS = 16
_NW = _NC * _NS
_RPW = _N // _NW
_CR = 1
_NCHUNK = _RPW // _CR
_NBUF = 8
_PF = 4


def _sc_body(x_hbm, v_hbm, out_hbm, buf, vals, in_sems, out_sems):
    wid = lax.axis_index("c") * _NS + lax.axis_index("s")
    r0 = wid * _RPW

    pltpu.make_async_copy(
        v_hbm.at[pl.ds(r0, _RPW)], vals.at[pl.ds(0, _RPW)], in_sems.at[0]
    ).start()
    pltpu.make_async_copy(
        v_hbm.at[pl.ds(r0, _RPW)], vals.at[pl.ds(0, _RPW)], in_sems.at[0]
    ).wait()

    def start_in(c, b):
        pltpu.make_async_copy(
            x_hbm.at[pl.ds(r0 + c * _CR, _CR), :], buf.at[b], in_sems.at[b]
        ).start()

    def wait_in(b):
        pltpu.make_async_copy(
            x_hbm.at[pl.ds(r0, _CR), :], buf.at[b], in_sems.at[b]
        ).wait()

    def start_out(c, b):
        pltpu.make_async_copy(
            buf.at[b], out_hbm.at[pl.ds(r0 + c * _CR, _CR), :], out_sems.at[b]
        ).start()

    def wait_out(b):
        pltpu.make_async_copy(
            buf.at[b], out_hbm.at[pl.ds(r0, _CR), :], out_sems.at[b]
        ).wait()

    for b in range(_PF):
        start_in(b, b)

    iota = lax.broadcasted_iota(jnp.int32, (16,), 0)
    diag_mask = iota < _CR

    def outer(o, _):
        for b in range(_NBUF):
            c = o * _NBUF + b
            wait_in(b)
            vals_v = plsc.load_gather(vals, [c * _CR + iota])
            col0 = r0 + c * _CR
            plsc.addupdate_scatter(
                buf.at[b], [iota, col0 + iota], vals_v, mask=diag_mask
            )
            start_out(c, b)
            nb = (b + _PF) % _NBUF

            @pl.when(c + _PF < _NCHUNK)
            def _():
                @pl.when(c + _PF >= _NBUF)
                def _():
                    wait_out(nb)

                start_in(c + _PF, nb)

        return ()

    lax.fori_loop(0, _NCHUNK // _NBUF, outer, ())

    for b in range(_NBUF):
        wait_out(b)


def kernel(x, values):
    mesh = plsc.VectorSubcoreMesh(
        core_axis_name="c", subcore_axis_name="s", num_cores=_NC, num_subcores=_NS
    )
    f = pl.kernel(
        _sc_body,
        out_type=jax.ShapeDtypeStruct((_N, _N), jnp.float32),
        mesh=mesh,
        scratch_types=[
            pltpu.VMEM((_NBUF, _CR, _N), jnp.float32),
            pltpu.VMEM((_RPW + 16, ), jnp.float32),
            pltpu.SemaphoreType.DMA((_NBUF,)),
            pltpu.SemaphoreType.DMA((_NBUF,)),
        ],
        compiler_params=pltpu.CompilerParams(needs_layout_passes=False),
    )
    return f(x, values)
